# baseline (speedup 1.0000x reference)
import functools

import jax
import jax.numpy as jnp
from jax import lax
from jax.experimental import pallas as pl
from jax.experimental.pallas import tpu as pltpu
from jax.experimental.pallas import tpu_sc as plsc

BATCH = 16384
EMBED_DIM = 32
N_ROWS = 1000000
_INFO = plsc.get_sparse_core_info()
NUM_WORKERS = _INFO.num_cores * _INFO.num_subcores
PER_WORKER = BATCH // NUM_WORKERS
CHUNK = 128
N_CHUNKS = PER_WORKER // CHUNK
CGROUPS = CHUNK // 16


def _mf_body(u_idx_hbm, i_idx_hbm, gm_hbm, ub_hbm, ib_hbm, ue_hbm, ie_hbm,
             out_hbm, uidx_v, iidx_v, uq_idx, iq_idx, ue_buf, ie_buf,
             ub_buf, ib_buf, gm_v, out_v, sem_u, sem_i, sem_ub, sem_ib):
    wid = lax.axis_index("s") * _INFO.num_cores + lax.axis_index("c")
    base = wid * PER_WORKER

    pltpu.sync_copy(u_idx_hbm.at[pl.ds(base, PER_WORKER)], uidx_v)
    pltpu.sync_copy(i_idx_hbm.at[pl.ds(base, PER_WORKER)], iidx_v)
    pltpu.sync_copy(gm_hbm, gm_v.at[pl.ds(0, 1)])

    gm = gm_v[...][0]
    lanes = lax.iota(jnp.int32, 16)

    for g in range(PER_WORKER // 16):
        j, k = g // CGROUPS, (g % CGROUPS) * 16
        uq_idx[j, k:k + 16] = uidx_v[pl.ds(g * 16, 16)] >> 2
        iq_idx[j, k:k + 16] = iidx_v[pl.ds(g * 16, 16)] >> 2

    def chunk_step(c, _):
        cu = pltpu.async_copy(ue_hbm.at[uq_idx.at[c]], ue_buf, sem_u)
        ci = pltpu.async_copy(ie_hbm.at[iq_idx.at[c]], ie_buf, sem_i)
        for lg in range(CGROUPS):
            iv_u = uidx_v[pl.ds(c * CHUNK + lg * 16, 16)]
            iv_i = iidx_v[pl.ds(c * CHUNK + lg * 16, 16)]
            for l in range(16):
                lb = lg * 16 + l
                q_u = iv_u[l] >> 3
                q_i = iv_i[l] >> 3
                pltpu.async_copy(ub_hbm.at[pl.ds(q_u, 1), :],
                                 ub_buf.at[pl.ds(lb, 1), :], sem_ub)
                pltpu.async_copy(ib_hbm.at[pl.ds(q_i, 1), :],
                                 ib_buf.at[pl.ds(lb, 1), :], sem_ib)
        cu.wait()
        ci.wait()
        pltpu.make_async_copy(ub_hbm.at[pl.ds(0, CHUNK), :], ub_buf,
                              sem_ub).wait()
        pltpu.make_async_copy(ib_hbm.at[pl.ds(0, CHUNK), :], ib_buf,
                              sem_ib).wait()

        for lg in range(CGROUPS):
            row = lanes + lg * 16
            iv_u = uidx_v[pl.ds(c * CHUNK + lg * 16, 16)]
            iv_i = iidx_v[pl.ds(c * CHUNK + lg * 16, 16)]
            col_u = (iv_u & 3) * EMBED_DIM
            col_i = (iv_i & 3) * EMBED_DIM
            acc = jnp.zeros((16,), jnp.float32)
            for d in range(EMBED_DIM):
                u = plsc.load_gather(ue_buf, [row, col_u + d])
                v = plsc.load_gather(ie_buf, [row, col_i + d])
                acc = acc + u * v
            ub = plsc.load_gather(ub_buf, [row, iv_u & 7])
            ib = plsc.load_gather(ib_buf, [row, iv_i & 7])
            out_v[pl.ds(c * CHUNK + lg * 16, 16)] = acc + ub + ib + gm
        return _

    lax.fori_loop(0, N_CHUNKS, chunk_step, None)

    pltpu.sync_copy(out_v, out_hbm.at[pl.ds(base, PER_WORKER)])


@jax.jit
def _mf_kernel(user_indices, item_indices, global_mean, user_bias, item_bias,
               user_embeddings, item_embeddings):
    mesh = plsc.VectorSubcoreMesh(core_axis_name="c", subcore_axis_name="s")
    ue2 = user_embeddings.reshape(N_ROWS // 4, 128)
    ie2 = item_embeddings.reshape(N_ROWS // 4, 128)
    ub2 = user_bias.reshape(N_ROWS // 8, 8)
    ib2 = item_bias.reshape(N_ROWS // 8, 8)
    return pl.kernel(
        _mf_body,
        mesh=mesh,
        compiler_params=pltpu.CompilerParams(needs_layout_passes=False),
        out_type=jax.ShapeDtypeStruct((BATCH,), jnp.float32),
        scratch_types=[
            pltpu.VMEM((PER_WORKER,), jnp.int32),
            pltpu.VMEM((PER_WORKER,), jnp.int32),
            pltpu.VMEM((N_CHUNKS, CHUNK), jnp.int32),
            pltpu.VMEM((N_CHUNKS, CHUNK), jnp.int32),
            pltpu.VMEM((CHUNK, 128), jnp.float32),
            pltpu.VMEM((CHUNK, 128), jnp.float32),
            pltpu.VMEM((CHUNK, 8), jnp.float32),
            pltpu.VMEM((CHUNK, 8), jnp.float32),
            pltpu.VMEM((16,), jnp.float32),
            pltpu.VMEM((PER_WORKER,), jnp.float32),
            pltpu.SemaphoreType.DMA,
            pltpu.SemaphoreType.DMA,
            pltpu.SemaphoreType.DMA,
            pltpu.SemaphoreType.DMA,
        ],
    )(user_indices, item_indices, global_mean, ub2, ib2, ue2, ie2)


def kernel(user_indices, item_indices, global_mean, user_bias, item_bias,
           user_embeddings, item_embeddings):
    return _mf_kernel(
        user_indices.astype(jnp.int32), item_indices.astype(jnp.int32),
        global_mean, user_bias, item_bias, user_embeddings, item_embeddings)


# speedup vs baseline: 1.0825x; 1.0825x over previous
import functools

import jax
import jax.numpy as jnp
from jax import lax
from jax.experimental import pallas as pl
from jax.experimental.pallas import tpu as pltpu
from jax.experimental.pallas import tpu_sc as plsc

BATCH = 16384
EMBED_DIM = 32
N_ROWS = 1000000
_INFO = plsc.get_sparse_core_info()
NUM_WORKERS = _INFO.num_cores * _INFO.num_subcores
PER_WORKER = BATCH // NUM_WORKERS
CHUNK = 128
N_CHUNKS = PER_WORKER // CHUNK
CGROUPS = CHUNK // 16


def _mf_body(u_idx_hbm, i_idx_hbm, gm_hbm, ub_hbm, ib_hbm, ue_hbm, ie_hbm,
             out_hbm, uidx_v, iidx_v, uq_idx, iq_idx, ue_buf, ie_buf,
             ub_buf, ib_buf, gm_v, out_v, sem_u, sem_i, sem_ub, sem_ib):
    wid = lax.axis_index("s") * _INFO.num_cores + lax.axis_index("c")
    base = wid * PER_WORKER

    pltpu.sync_copy(u_idx_hbm.at[pl.ds(base, PER_WORKER)], uidx_v)
    pltpu.sync_copy(i_idx_hbm.at[pl.ds(base, PER_WORKER)], iidx_v)
    pltpu.sync_copy(gm_hbm, gm_v.at[pl.ds(0, 1)])

    gm = gm_v[...][0]
    lanes = lax.iota(jnp.int32, 16)

    for g in range(PER_WORKER // 16):
        j, k = g // CGROUPS, (g % CGROUPS) * 16
        uq_idx[j, k:k + 16] = uidx_v[pl.ds(g * 16, 16)] >> 2
        iq_idx[j, k:k + 16] = iidx_v[pl.ds(g * 16, 16)] >> 2

    def chunk_step(c, _):
        cu = pltpu.async_copy(ue_hbm.at[uq_idx.at[c]], ue_buf, sem_u)
        ci = pltpu.async_copy(ie_hbm.at[iq_idx.at[c]], ie_buf, sem_i)
        for lg in range(CGROUPS):
            iv_u = uidx_v[pl.ds(c * CHUNK + lg * 16, 16)]
            iv_i = iidx_v[pl.ds(c * CHUNK + lg * 16, 16)]
            for l in range(16):
                lb = lg * 16 + l
                a_u = pl.multiple_of(iv_u[l] & -8, 8)
                a_i = pl.multiple_of(iv_i[l] & -8, 8)
                pltpu.async_copy(ub_hbm.at[pl.ds(a_u, 8)],
                                 ub_buf.at[pl.ds(lb * 8, 8)], sem_ub)
                pltpu.async_copy(ib_hbm.at[pl.ds(a_i, 8)],
                                 ib_buf.at[pl.ds(lb * 8, 8)], sem_ib)
        cu.wait()
        ci.wait()
        pltpu.make_async_copy(ub_hbm.at[pl.ds(0, CHUNK * 8)], ub_buf,
                              sem_ub).wait()
        pltpu.make_async_copy(ib_hbm.at[pl.ds(0, CHUNK * 8)], ib_buf,
                              sem_ib).wait()

        for lg in range(CGROUPS):
            row = lanes + lg * 16
            iv_u = uidx_v[pl.ds(c * CHUNK + lg * 16, 16)]
            iv_i = iidx_v[pl.ds(c * CHUNK + lg * 16, 16)]
            col_u = (iv_u & 3) * EMBED_DIM
            col_i = (iv_i & 3) * EMBED_DIM
            acc = jnp.zeros((16,), jnp.float32)
            for d in range(EMBED_DIM):
                u = plsc.load_gather(ue_buf, [row, col_u + d])
                v = plsc.load_gather(ie_buf, [row, col_i + d])
                acc = acc + u * v
            w_u = row * 8 + (iv_u & 7)
            w_i = row * 8 + (iv_i & 7)
            ub = plsc.load_gather(ub_buf, [w_u])
            ib = plsc.load_gather(ib_buf, [w_i])
            out_v[pl.ds(c * CHUNK + lg * 16, 16)] = acc + ub + ib + gm
        return _

    lax.fori_loop(0, N_CHUNKS, chunk_step, None)

    pltpu.sync_copy(out_v, out_hbm.at[pl.ds(base, PER_WORKER)])


@jax.jit
def _mf_kernel(user_indices, item_indices, global_mean, user_bias, item_bias,
               user_embeddings, item_embeddings):
    mesh = plsc.VectorSubcoreMesh(core_axis_name="c", subcore_axis_name="s")
    ue2 = user_embeddings.reshape(N_ROWS // 4, 128)
    ie2 = item_embeddings.reshape(N_ROWS // 4, 128)
    ub1 = user_bias[:, 0]
    ib1 = item_bias[:, 0]
    return pl.kernel(
        _mf_body,
        mesh=mesh,
        compiler_params=pltpu.CompilerParams(needs_layout_passes=False),
        out_type=jax.ShapeDtypeStruct((BATCH,), jnp.float32),
        scratch_types=[
            pltpu.VMEM((PER_WORKER,), jnp.int32),
            pltpu.VMEM((PER_WORKER,), jnp.int32),
            pltpu.VMEM((N_CHUNKS, CHUNK), jnp.int32),
            pltpu.VMEM((N_CHUNKS, CHUNK), jnp.int32),
            pltpu.VMEM((CHUNK, 128), jnp.float32),
            pltpu.VMEM((CHUNK, 128), jnp.float32),
            pltpu.VMEM((CHUNK * 8,), jnp.float32),
            pltpu.VMEM((CHUNK * 8,), jnp.float32),
            pltpu.VMEM((16,), jnp.float32),
            pltpu.VMEM((PER_WORKER,), jnp.float32),
            pltpu.SemaphoreType.DMA,
            pltpu.SemaphoreType.DMA,
            pltpu.SemaphoreType.DMA,
            pltpu.SemaphoreType.DMA,
        ],
    )(user_indices, item_indices, global_mean, ub1, ib1, ue2, ie2)


def kernel(user_indices, item_indices, global_mean, user_bias, item_bias,
           user_embeddings, item_embeddings):
    return _mf_kernel(
        user_indices.astype(jnp.int32), item_indices.astype(jnp.int32),
        global_mean, user_bias, item_bias, user_embeddings, item_embeddings)
